# Initial kernel scaffold; baseline (speedup 1.0000x reference)
import jax
import jax.numpy as jnp
from jax.experimental import pallas as pl


def kernel(in_feat, edge_index, W1, b1, W2, b2):
    raise NotImplementedError("write your pallas kernel here")


# speedup vs baseline: 4.4786x; 4.4786x over previous
import functools

import jax
import jax.numpy as jnp
from jax import lax
from jax.experimental import pallas as pl
from jax.experimental.pallas import tpu as pltpu
from jax.experimental.pallas import tpu_sc as plsc

N = 10000
E = 320000
F = 128
C = 40
CP = 48
NP = 10240
TRASH = N
NC = 2
NS = 16
NW = NC * NS
CHUNK = 128
CHUNKS = 80
EPAD = NW * CHUNKS * CHUNK
assert EPAD >= E
FH = F // 2
RPT = NP // NS


def _sc_mesh():
    return plsc.VectorSubcoreMesh(core_axis_name="c", subcore_axis_name="s")


_SC_PARAMS = pltpu.CompilerParams(use_tc_tiling_on_sc=False)


@functools.partial(
    pl.kernel,
    out_type=(
        jax.ShapeDtypeStruct((NC, NP, 16), jnp.float32),
        jax.ShapeDtypeStruct((NC, NP, 16), jnp.float32),
    ),
    mesh=_sc_mesh(),
    compiler_params=_SC_PARAMS,
    scratch_types=[
        pltpu.VMEM((CHUNKS, CHUNK), jnp.int32),
        pltpu.VMEM((CHUNKS, CHUNK), jnp.int32),
        pltpu.VMEM((CHUNK, 16), jnp.float32),
        pltpu.VMEM_SHARED((NP, 16), jnp.float32),
        pltpu.VMEM_SHARED((NP, 16), jnp.float32),
        pltpu.SemaphoreType.DMA,
        pltpu.SemaphoreType.DMA,
    ],
)
def _sc_degrees(src_hbm, dst_hbm, ones_hbm, zeros_hbm,
                osrc_hbm, odst_hbm,
                src_v, dst_v, ones_v, dsrc, ddst, sem_a, sem_b):
    c = lax.axis_index("c")
    s = lax.axis_index("s")
    wid = s * NC + c
    pltpu.sync_copy(src_hbm.at[wid], src_v)
    pltpu.sync_copy(dst_hbm.at[wid], dst_v)
    pltpu.sync_copy(ones_hbm, ones_v)
    base = s * RPT
    pltpu.sync_copy(zeros_hbm, dsrc.at[pl.ds(base, RPT)])
    pltpu.sync_copy(zeros_hbm, ddst.at[pl.ds(base, RPT)])
    plsc.subcore_barrier()

    def body(j, carry):
        a = pltpu.async_copy(ones_v, dsrc.at[src_v.at[j]], sem_a, add=True)
        b = pltpu.async_copy(ones_v, ddst.at[dst_v.at[j]], sem_b, add=True)
        a.wait()
        b.wait()
        return carry

    lax.fori_loop(0, CHUNKS, body, 0)
    plsc.subcore_barrier()
    pltpu.sync_copy(dsrc.at[pl.ds(base, RPT)], osrc_hbm.at[c, pl.ds(base, RPT)])
    pltpu.sync_copy(ddst.at[pl.ds(base, RPT)], odst_hbm.at[c, pl.ds(base, RPT)])


def _make_propagate(w, parts):
    out_t = tuple(
        jax.ShapeDtypeStruct((NC, NP, w), jnp.float32) for _ in range(parts)
    )

    @functools.partial(
        pl.kernel,
        out_type=out_t,
        mesh=_sc_mesh(),
        compiler_params=_SC_PARAMS,
        scratch_types=[
            pltpu.VMEM((CHUNKS, CHUNK), jnp.int32),
            pltpu.VMEM((CHUNKS, CHUNK), jnp.int32),
            pltpu.VMEM((CHUNK, w), jnp.float32),
            pltpu.VMEM((CHUNK, w), jnp.float32),
            pltpu.VMEM_SHARED((NP, w), jnp.float32),
            pltpu.SemaphoreType.DMA,
            pltpu.SemaphoreType.DMA,
        ],
    )
    def prop(src_hbm, dst_hbm, *rest):
        h_list = rest[:parts]
        zeros_hbm = rest[parts]
        out_list = rest[parts + 1:2 * parts + 1]
        src_v, dst_v, buf0, buf1, agg, gsem, ssem = rest[2 * parts + 1:]
        c = lax.axis_index("c")
        s = lax.axis_index("s")
        wid = s * NC + c
        pltpu.sync_copy(src_hbm.at[wid], src_v)
        pltpu.sync_copy(dst_hbm.at[wid], dst_v)
        base = s * RPT
        for part in range(parts):
            h_hbm = h_list[part]
            pltpu.sync_copy(zeros_hbm, agg.at[pl.ds(base, RPT)])
            plsc.subcore_barrier()

            pltpu.async_copy(h_hbm.at[src_v.at[0]], buf0, gsem)

            def pair_body(p, carry, h_hbm=h_hbm):
                j0 = p * 2
                pltpu.make_async_copy(h_hbm.at[src_v.at[j0]], buf0, gsem).wait()
                g1 = pltpu.async_copy(h_hbm.at[src_v.at[j0 + 1]], buf1, gsem)
                pltpu.async_copy(buf0, agg.at[dst_v.at[j0]], ssem, add=True).wait()
                g1.wait()

                @pl.when(j0 + 2 < CHUNKS)
                def _():
                    pltpu.async_copy(h_hbm.at[src_v.at[j0 + 2]], buf0, gsem)

                pltpu.async_copy(buf1, agg.at[dst_v.at[j0 + 1]], ssem,
                                 add=True).wait()
                return carry

            lax.fori_loop(0, CHUNKS // 2, pair_body, 0)
            plsc.subcore_barrier()
            pltpu.sync_copy(agg.at[pl.ds(base, RPT)],
                            out_list[part].at[c, pl.ds(base, RPT)])

    return prop


_sc_prop_f = _make_propagate(FH, 2)
_sc_prop_c = _make_propagate(CP, 1)


_RB = 1280


def _k2_body(dsrc_ref, ddst_ref, x_ref, w1_ref, on_ref, in_ref, h0a_ref,
             h0b_ref):
    ds = jnp.max(dsrc_ref[0] + dsrc_ref[1], axis=1, keepdims=True)
    di = jnp.max(ddst_ref[0] + ddst_ref[1], axis=1, keepdims=True)
    on = lax.rsqrt(jnp.where(ds > 0, ds, 1.0))
    inn = lax.rsqrt(jnp.where(di > 0, di, 1.0))
    on_ref[...] = on
    in_ref[...] = inn
    xw = jnp.dot(x_ref[...], w1_ref[...], preferred_element_type=jnp.float32)
    h0a_ref[...] = xw[:, :FH] * on
    h0b_ref[...] = xw[:, FH:] * on


def _tc_norms_h0(dsrc, ddst, x_p, w1):
    return pl.pallas_call(
        _k2_body,
        grid=(NP // _RB,),
        in_specs=[
            pl.BlockSpec((NC, _RB, 16), lambda i: (0, i, 0)),
            pl.BlockSpec((NC, _RB, 16), lambda i: (0, i, 0)),
            pl.BlockSpec((_RB, F), lambda i: (i, 0)),
            pl.BlockSpec((F, F), lambda i: (0, 0)),
        ],
        out_specs=[
            pl.BlockSpec((_RB, 1), lambda i: (i, 0)),
            pl.BlockSpec((_RB, 1), lambda i: (i, 0)),
            pl.BlockSpec((_RB, FH), lambda i: (i, 0)),
            pl.BlockSpec((_RB, FH), lambda i: (i, 0)),
        ],
        out_shape=[
            jax.ShapeDtypeStruct((NP, 1), jnp.float32),
            jax.ShapeDtypeStruct((NP, 1), jnp.float32),
            jax.ShapeDtypeStruct((NP, FH), jnp.float32),
            jax.ShapeDtypeStruct((NP, FH), jnp.float32),
        ],
    )(dsrc, ddst, x_p, w1)


def _k4_body(s1a_ref, s1b_ref, in_ref, on_ref, b1_ref, w2_ref, t_ref):
    inn = in_ref[...]
    onn = on_ref[...]
    b1v = b1_ref[...]
    a = (s1a_ref[0] + s1a_ref[1]) * inn
    b = (s1b_ref[0] + s1b_ref[1]) * inn
    h1a = jnp.maximum(a + b1v[None, :FH], 0.0) * onn
    h1b = jnp.maximum(b + b1v[None, FH:], 0.0) * onn
    t_ref[...] = (
        jnp.dot(h1a, w2_ref[:FH, :], preferred_element_type=jnp.float32)
        + jnp.dot(h1b, w2_ref[FH:, :], preferred_element_type=jnp.float32)
    )


def _tc_layer2(s1a, s1b, inn, onn, b1, w2p):
    return pl.pallas_call(
        _k4_body,
        grid=(NP // _RB,),
        in_specs=[
            pl.BlockSpec((NC, _RB, FH), lambda i: (0, i, 0)),
            pl.BlockSpec((NC, _RB, FH), lambda i: (0, i, 0)),
            pl.BlockSpec((_RB, 1), lambda i: (i, 0)),
            pl.BlockSpec((_RB, 1), lambda i: (i, 0)),
            pl.BlockSpec((F,), lambda i: (0,)),
            pl.BlockSpec((F, CP), lambda i: (0, 0)),
        ],
        out_specs=pl.BlockSpec((_RB, CP), lambda i: (i, 0)),
        out_shape=jax.ShapeDtypeStruct((NP, CP), jnp.float32),
    )(s1a, s1b, inn, onn, b1, w2p)


_RB6 = 2000


def _k6_body(s2_ref, in_ref, b2_ref, o_ref):
    logits = (s2_ref[0] + s2_ref[1]) * in_ref[...] + b2_ref[...][None, :]
    col = lax.broadcasted_iota(jnp.int32, (_RB6, CP), 1)
    lm = jnp.where(col < C, logits, -1e30)
    m = jnp.max(lm, axis=-1, keepdims=True)
    lse = jnp.log(jnp.sum(jnp.exp(lm - m), axis=-1, keepdims=True))
    o_ref[...] = (logits - m - lse)[:, :C]


def _tc_final(s2, inn, b2p):
    return pl.pallas_call(
        _k6_body,
        grid=(N // _RB6,),
        in_specs=[
            pl.BlockSpec((NC, _RB6, CP), lambda i: (0, i, 0)),
            pl.BlockSpec((_RB6, 1), lambda i: (i, 0)),
            pl.BlockSpec((CP,), lambda i: (0,)),
        ],
        out_specs=pl.BlockSpec((_RB6, C), lambda i: (i, 0)),
        out_shape=jax.ShapeDtypeStruct((N, C), jnp.float32),
    )(s2, inn, b2p)


def kernel(in_feat, edge_index, W1, b1, W2, b2):
    src = edge_index[0]
    dst = edge_index[1]
    fill = jnp.full((EPAD - E,), TRASH, jnp.int32)
    src_p = jnp.concatenate([src, fill]).reshape(NW, CHUNKS, CHUNK)
    dst_p = jnp.concatenate([dst, fill]).reshape(NW, CHUNKS, CHUNK)
    x_p = jnp.pad(in_feat, ((0, NP - N), (0, 0)))
    w2p = jnp.pad(W2, ((0, 0), (0, CP - C)))
    b2p = jnp.pad(b2, (0, CP - C))

    ones16 = jnp.ones((CHUNK, 16), jnp.float32)
    zeros16 = jnp.zeros((RPT, 16), jnp.float32)
    zeros_f = jnp.zeros((RPT, FH), jnp.float32)
    zeros_c = jnp.zeros((RPT, CP), jnp.float32)

    dsrc, ddst = _sc_degrees(src_p, dst_p, ones16, zeros16)
    onn, inn, h0a, h0b = _tc_norms_h0(dsrc, ddst, x_p, W1)
    s1a, s1b = _sc_prop_f(src_p, dst_p, h0a, h0b, zeros_f)
    t = _tc_layer2(s1a, s1b, inn, onn, b1, w2p)
    (s2,) = _sc_prop_c(src_p, dst_p, t, zeros_c)
    return _tc_final(s2, inn, b2p)


# speedup vs baseline: 5.3838x; 1.2021x over previous
import functools

import jax
import jax.numpy as jnp
from jax import lax
from jax.experimental import pallas as pl
from jax.experimental.pallas import tpu as pltpu
from jax.experimental.pallas import tpu_sc as plsc

N = 10000
E = 320000
F = 128
C = 40
CP = 48
NP = 10240
TRASH = N
NC = 2
NS = 16
NW = NC * NS
EPW = 10176
EPAD = NW * EPW
assert EPAD >= E
RPT = NP // NS


def _sc_mesh():
    return plsc.VectorSubcoreMesh(core_axis_name="c", subcore_axis_name="s")


_SC_PARAMS = pltpu.CompilerParams(use_tc_tiling_on_sc=False)


def _make_propagate(w, chunk, chunks, nbuf):
    @functools.partial(
        pl.kernel,
        out_type=jax.ShapeDtypeStruct((NC, NP, w), jnp.float32),
        mesh=_sc_mesh(),
        compiler_params=_SC_PARAMS,
        scratch_types=[
            pltpu.VMEM((chunks, chunk), jnp.int32),
            pltpu.VMEM((chunks, chunk), jnp.int32),
            [pltpu.VMEM((chunk, w), jnp.float32)] * nbuf,
            pltpu.VMEM_SHARED((NP, w), jnp.float32),
            pltpu.SemaphoreType.DMA,
            pltpu.SemaphoreType.DMA,
        ],
    )
    def prop(src_hbm, dst_hbm, h_hbm, zeros_hbm, out_hbm,
             src_v, dst_v, bufs, agg, gsem, ssem):
        c = lax.axis_index("c")
        s = lax.axis_index("s")
        wid = s * NC + c
        pltpu.sync_copy(src_hbm.at[wid], src_v)
        pltpu.sync_copy(dst_hbm.at[wid], dst_v)
        base = s * RPT
        pltpu.sync_copy(zeros_hbm, agg.at[pl.ds(base, RPT)])
        plsc.subcore_barrier()

        for q in range(nbuf - 1):
            pltpu.async_copy(h_hbm.at[src_v.at[q]], bufs[q], gsem)

        def step(q, b, wait_prev, issue_next):
            pltpu.make_async_copy(h_hbm.at[src_v.at[q]], bufs[b], gsem).wait()
            pltpu.async_copy(bufs[b], agg.at[dst_v.at[q]], ssem, add=True)
            prev = (b + nbuf - 1) % nbuf

            def _wait_prev():
                pltpu.make_async_copy(
                    bufs[prev], agg.at[dst_v.at[q - 1]], ssem).wait()

            def _issue_next():
                pltpu.async_copy(
                    h_hbm.at[src_v.at[q + nbuf - 1]], bufs[prev], gsem)

            if wait_prev is True:
                _wait_prev()
            else:
                pl.when(wait_prev)(_wait_prev)
            if issue_next is True:
                _issue_next()
            elif issue_next is not False:
                pl.when(issue_next)(_issue_next)

        n_full = chunks // nbuf
        tail = chunks - n_full * nbuf

        def ring_body(p, carry):
            q0 = p * nbuf
            for b in range(nbuf):
                q = q0 + b
                step(q, b,
                     wait_prev=True if b > 0 else q > 0,
                     issue_next=q + nbuf - 1 < chunks)
            return carry

        lax.fori_loop(0, n_full, ring_body, 0)
        for q in range(chunks - tail, chunks):
            step(q, q % nbuf, wait_prev=True, issue_next=False)
        pltpu.make_async_copy(
            bufs[(chunks - 1) % nbuf], agg.at[dst_v.at[chunks - 1]], ssem
        ).wait()

        plsc.subcore_barrier()
        pltpu.sync_copy(agg.at[pl.ds(base, RPT)],
                        out_hbm.at[c, pl.ds(base, RPT)])

    return prop


_C3, _N3 = 64, 159
_C5, _N5 = 96, 106
_sc_prop_f = _make_propagate(F, _C3, _N3, 3)
_sc_prop_c = _make_propagate(CP, _C5, _N5, 4)


@functools.partial(
    pl.kernel,
    out_type=(
        jax.ShapeDtypeStruct((NC, NP, 16), jnp.float32),
        jax.ShapeDtypeStruct((NC, NP, 16), jnp.float32),
    ),
    mesh=_sc_mesh(),
    compiler_params=_SC_PARAMS,
    scratch_types=[
        pltpu.VMEM((_N5, _C5), jnp.int32),
        pltpu.VMEM((_N5, _C5), jnp.int32),
        pltpu.VMEM((_C5, 16), jnp.float32),
        pltpu.VMEM_SHARED((NP, 16), jnp.float32),
        pltpu.VMEM_SHARED((NP, 16), jnp.float32),
        pltpu.SemaphoreType.DMA,
        pltpu.SemaphoreType.DMA,
    ],
)
def _sc_degrees(src_hbm, dst_hbm, ones_hbm, zeros_hbm,
                osrc_hbm, odst_hbm,
                src_v, dst_v, ones_v, dsrc, ddst, sem_a, sem_b):
    c = lax.axis_index("c")
    s = lax.axis_index("s")
    wid = s * NC + c
    pltpu.sync_copy(src_hbm.at[wid], src_v)
    pltpu.sync_copy(dst_hbm.at[wid], dst_v)
    pltpu.sync_copy(ones_hbm, ones_v)
    base = s * RPT
    pltpu.sync_copy(zeros_hbm, dsrc.at[pl.ds(base, RPT)])
    pltpu.sync_copy(zeros_hbm, ddst.at[pl.ds(base, RPT)])
    plsc.subcore_barrier()

    def body(j, carry):
        a = pltpu.async_copy(ones_v, dsrc.at[src_v.at[j]], sem_a, add=True)
        b = pltpu.async_copy(ones_v, ddst.at[dst_v.at[j]], sem_b, add=True)
        a.wait()
        b.wait()
        return carry

    lax.fori_loop(0, _N5, body, 0)
    plsc.subcore_barrier()
    pltpu.sync_copy(dsrc.at[pl.ds(base, RPT)], osrc_hbm.at[c, pl.ds(base, RPT)])
    pltpu.sync_copy(ddst.at[pl.ds(base, RPT)], odst_hbm.at[c, pl.ds(base, RPT)])


_RB = 1280


def _k2_body(dsrc_ref, ddst_ref, x_ref, w1_ref, on_ref, in_ref, h0_ref):
    ds = jnp.max(dsrc_ref[0] + dsrc_ref[1], axis=1, keepdims=True)
    di = jnp.max(ddst_ref[0] + ddst_ref[1], axis=1, keepdims=True)
    on = lax.rsqrt(jnp.where(ds > 0, ds, 1.0))
    inn = lax.rsqrt(jnp.where(di > 0, di, 1.0))
    on_ref[...] = on
    in_ref[...] = inn
    xw = jnp.dot(x_ref[...], w1_ref[...], preferred_element_type=jnp.float32)
    h0_ref[...] = xw * on


def _tc_norms_h0(dsrc, ddst, x_p, w1):
    return pl.pallas_call(
        _k2_body,
        grid=(NP // _RB,),
        in_specs=[
            pl.BlockSpec((NC, _RB, 16), lambda i: (0, i, 0)),
            pl.BlockSpec((NC, _RB, 16), lambda i: (0, i, 0)),
            pl.BlockSpec((_RB, F), lambda i: (i, 0)),
            pl.BlockSpec((F, F), lambda i: (0, 0)),
        ],
        out_specs=[
            pl.BlockSpec((_RB, 1), lambda i: (i, 0)),
            pl.BlockSpec((_RB, 1), lambda i: (i, 0)),
            pl.BlockSpec((_RB, F), lambda i: (i, 0)),
        ],
        out_shape=[
            jax.ShapeDtypeStruct((NP, 1), jnp.float32),
            jax.ShapeDtypeStruct((NP, 1), jnp.float32),
            jax.ShapeDtypeStruct((NP, F), jnp.float32),
        ],
    )(dsrc, ddst, x_p, w1)


def _k4_body(s1_ref, in_ref, on_ref, b1_ref, w2_ref, t_ref):
    agg = s1_ref[0] + s1_ref[1]
    h1 = jnp.maximum(agg * in_ref[...] + b1_ref[...][None, :], 0.0)
    h1 = h1 * on_ref[...]
    t_ref[...] = jnp.dot(h1, w2_ref[...], preferred_element_type=jnp.float32)


def _tc_layer2(s1, inn, onn, b1, w2p):
    return pl.pallas_call(
        _k4_body,
        grid=(NP // _RB,),
        in_specs=[
            pl.BlockSpec((NC, _RB, F), lambda i: (0, i, 0)),
            pl.BlockSpec((_RB, 1), lambda i: (i, 0)),
            pl.BlockSpec((_RB, 1), lambda i: (i, 0)),
            pl.BlockSpec((F,), lambda i: (0,)),
            pl.BlockSpec((F, CP), lambda i: (0, 0)),
        ],
        out_specs=pl.BlockSpec((_RB, CP), lambda i: (i, 0)),
        out_shape=jax.ShapeDtypeStruct((NP, CP), jnp.float32),
    )(s1, inn, onn, b1, w2p)


_RB6 = 2000


def _k6_body(s2_ref, in_ref, b2_ref, o_ref):
    logits = (s2_ref[0] + s2_ref[1]) * in_ref[...] + b2_ref[...][None, :]
    col = lax.broadcasted_iota(jnp.int32, (_RB6, CP), 1)
    lm = jnp.where(col < C, logits, -1e30)
    m = jnp.max(lm, axis=-1, keepdims=True)
    lse = jnp.log(jnp.sum(jnp.exp(lm - m), axis=-1, keepdims=True))
    o_ref[...] = (logits - m - lse)[:, :C]


def _tc_final(s2, inn, b2p):
    return pl.pallas_call(
        _k6_body,
        grid=(N // _RB6,),
        in_specs=[
            pl.BlockSpec((NC, _RB6, CP), lambda i: (0, i, 0)),
            pl.BlockSpec((_RB6, 1), lambda i: (i, 0)),
            pl.BlockSpec((CP,), lambda i: (0,)),
        ],
        out_specs=pl.BlockSpec((_RB6, C), lambda i: (i, 0)),
        out_shape=jax.ShapeDtypeStruct((N, C), jnp.float32),
    )(s2, inn, b2p)


def kernel(in_feat, edge_index, W1, b1, W2, b2):
    src = edge_index[0]
    dst = edge_index[1]
    fill = jnp.full((EPAD - E,), TRASH, jnp.int32)
    src_flat = jnp.concatenate([src, fill])
    dst_flat = jnp.concatenate([dst, fill])
    src3 = src_flat.reshape(NW, _N3, _C3)
    dst3 = dst_flat.reshape(NW, _N3, _C3)
    src5 = src_flat.reshape(NW, _N5, _C5)
    dst5 = dst_flat.reshape(NW, _N5, _C5)
    x_p = jnp.pad(in_feat, ((0, NP - N), (0, 0)))
    w2p = jnp.pad(W2, ((0, 0), (0, CP - C)))
    b2p = jnp.pad(b2, (0, CP - C))

    ones16 = jnp.ones((_C5, 16), jnp.float32)
    zeros16 = jnp.zeros((RPT, 16), jnp.float32)
    zeros_f = jnp.zeros((RPT, F), jnp.float32)
    zeros_c = jnp.zeros((RPT, CP), jnp.float32)

    dsrc, ddst = _sc_degrees(src5, dst5, ones16, zeros16)
    onn, inn, h0 = _tc_norms_h0(dsrc, ddst, x_p, W1)
    s1 = _sc_prop_f(src3, dst3, h0, zeros_f)
    t = _tc_layer2(s1, inn, onn, b1, w2p)
    s2 = _sc_prop_c(src5, dst5, t, zeros_c)
    return _tc_final(s2, inn, b2p)
